# baseline (speedup 1.0000x reference)
import functools
import math

import jax
import jax.numpy as jnp
from jax import lax
from jax.experimental import pallas as pl
from jax.experimental.pallas import tpu as pltpu
from jax.experimental.pallas import tpu_sc as plsc

_VOCAB = 100000
_EMB = 128
_SCALE = math.sqrt(float(_EMB))

_B = 4096 * 200
_NW = 32
_BPW = _B // _NW
_C = 128
_NCHUNK = _BPW // _C
_NBF = 4
_NRW = 2
_W = _EMB // 2

_mesh = plsc.VectorSubcoreMesh(core_axis_name="c", subcore_axis_name="s")


@functools.partial(
    pl.kernel,
    mesh=_mesh,
    compiler_params=pltpu.CompilerParams(needs_layout_passes=False, use_tc_tiling_on_sc=False),
    out_type=jax.ShapeDtypeStruct((_B, _EMB), jnp.float32),
    scratch_types=[
        pltpu.VMEM((_NCHUNK, _C), jnp.int32),
        pltpu.VMEM((_NBF, _C, 2, _W), jnp.bfloat16),
        pltpu.VMEM((_NRW, _C, _EMB), jnp.float32),
        pltpu.SemaphoreType.DMA,
        pltpu.SemaphoreType.DMA,
    ],
)
def _gather(tokens_hbm, table_hbm, out_hbm, idx_v, pk_v, rows_v, gsem, ssem):
    cid = lax.axis_index("c")
    sid = lax.axis_index("s")
    wid = sid * 2 + cid
    base = wid * _BPW

    pltpu.sync_copy(tokens_hbm.at[wid], idx_v)

    def g_copy(g, b):
        return pltpu.make_async_copy(
            table_hbm.at[idx_v.at[g]], pk_v.at[b], gsem
        )

    def s_copy(g, rs):
        return pltpu.make_async_copy(
            rows_v.at[rs], out_hbm.at[pl.ds(base + g * _C, _C)], ssem
        )

    lanes = lax.iota(jnp.int32, 16)
    ev_cols = [lanes * 2 + 32 * j for j in range(_EMB // 32)]
    od_cols = [c + 1 for c in ev_cols]

    def expand(bs, rs):
        def erow(r, carry):
            rr = jnp.full((16,), r, jnp.int32)
            dst = rows_v.at[rs]
            for j in range(_EMB // 32):
                s, o = divmod(32 * j, _W)
                pairs = pk_v[bs, r, s, pl.ds(o, 32)]
                ev, od = plsc.unpack(pairs, format=plsc.PackFormat.INTERLEAVED)
                plsc.store_scatter(dst, [rr, ev_cols[j]], ev * _SCALE)
                plsc.store_scatter(dst, [rr, od_cols[j]], od * _SCALE)
            return carry

        lax.fori_loop(0, _C, erow, 0)

    def chunk(g, b, wait_s, issue_g):
        rs = b % _NRW
        g_copy(g, b).wait()
        if wait_s:
            s_copy(g - _NRW, rs).wait()
        expand(b, rs)
        s_copy(g, rs).start()
        if issue_g:
            g_copy(g + _NBF, b).start()

    for b in range(_NBF):
        g_copy(b, b).start()

    for b in range(_NBF):
        chunk(b, b, wait_s=(b >= _NRW), issue_g=True)

    def body(i, carry):
        g0 = i * _NBF
        for b in range(_NBF):
            chunk(g0 + b, b, wait_s=True, issue_g=True)
        return carry

    lax.fori_loop(1, _NCHUNK // _NBF - 1, body, 0)

    g0 = _NCHUNK - _NBF
    for b in range(_NBF):
        chunk(g0 + b, b, wait_s=True, issue_g=False)

    for g in range(_NCHUNK - _NRW, _NCHUNK):
        s_copy(g, g % _NRW).wait()


def kernel(tokens, table):
    tbl = table.astype(jnp.bfloat16).reshape(_VOCAB, 2, _W)
    tok = tokens.reshape(_NW, _NCHUNK, _C).astype(jnp.int32)
    out = _gather(tok, tbl)
    return out.reshape(tokens.shape[0], tokens.shape[1], _EMB)


# speedup vs baseline: 1.2062x; 1.2062x over previous
import functools
import math

import jax
import jax.numpy as jnp
from jax import lax
from jax.experimental import pallas as pl
from jax.experimental.pallas import tpu as pltpu
from jax.experimental.pallas import tpu_sc as plsc

_VOCAB = 100000
_EMB = 128
_SCALE = math.sqrt(float(_EMB))

_B = 4096 * 200
_NW = 32
_BPW = _B // _NW
_C = 128
_NCHUNK = _BPW // _C
_NBF = 4
_NRW = 2
_W = _EMB // 2

_mesh = plsc.VectorSubcoreMesh(core_axis_name="c", subcore_axis_name="s")


@functools.partial(
    pl.kernel,
    mesh=_mesh,
    compiler_params=pltpu.CompilerParams(
        needs_layout_passes=False, use_tc_tiling_on_sc=False
    ),
    out_type=jax.ShapeDtypeStruct((_B, _EMB), jnp.float32),
    scratch_types=[
        pltpu.VMEM((_NCHUNK, _C), jnp.int32),
        pltpu.VMEM((_NBF, _C, _W), jnp.int32),
        pltpu.VMEM((_NRW, _C, _EMB), jnp.float32),
        pltpu.SemaphoreType.DMA,
        pltpu.SemaphoreType.DMA,
    ],
)
def _gather(tokens_hbm, table_hbm, out_hbm, idx_v, pk_v, rows_v, gsem, ssem):
    cid = lax.axis_index("c")
    sid = lax.axis_index("s")
    wid = sid * 2 + cid
    base = wid * _BPW

    pltpu.sync_copy(tokens_hbm.at[wid], idx_v)

    def g_copy(g, b):
        return pltpu.make_async_copy(
            table_hbm.at[idx_v.at[g]], pk_v.at[b], gsem
        )

    def s_copy(g, rs):
        return pltpu.make_async_copy(
            rows_v.at[rs], out_hbm.at[pl.ds(base + g * _C, _C)], ssem
        )

    himask = jnp.full((16,), -65536, jnp.int32)

    def expand(bs, rs):
        def erow(r, carry):
            for j in range(_W // 16):
                w = pk_v[bs, r, pl.ds(16 * j, 16)]
                lo = plsc.bitcast(w << 16, jnp.float32) * _SCALE
                hi = plsc.bitcast(w & himask, jnp.float32) * _SCALE
                rows_v[rs, r, pl.ds(16 * j, 16)] = lo
                rows_v[rs, r, pl.ds(_W + 16 * j, 16)] = hi
            return carry

        lax.fori_loop(0, _C, erow, 0)

    def chunk(g, b, wait_s, issue_g):
        rs = b % _NRW
        g_copy(g, b).wait()
        if wait_s:
            s_copy(g - _NRW, rs).wait()
        expand(b, rs)
        s_copy(g, rs).start()
        if issue_g:
            g_copy(g + _NBF, b).start()

    for b in range(_NBF):
        g_copy(b, b).start()

    for b in range(_NBF):
        chunk(b, b, wait_s=(b >= _NRW), issue_g=True)

    def body(i, carry):
        g0 = i * _NBF
        for b in range(_NBF):
            chunk(g0 + b, b, wait_s=True, issue_g=True)
        return carry

    lax.fori_loop(1, _NCHUNK // _NBF - 1, body, 0)

    g0 = _NCHUNK - _NBF
    for b in range(_NBF):
        chunk(g0 + b, b, wait_s=True, issue_g=False)

    for g in range(_NCHUNK - _NRW, _NCHUNK):
        s_copy(g, g % _NRW).wait()


def kernel(tokens, table):
    tb = table.astype(jnp.bfloat16)
    pairs = jnp.stack([tb[:, :_W], tb[:, _W:]], axis=-1)
    tbl = lax.bitcast_convert_type(pairs, jnp.int32)
    tok = tokens.reshape(_NW, _NCHUNK, _C).astype(jnp.int32)
    out = _gather(tok, tbl)
    return out.reshape(tokens.shape[0], tokens.shape[1], _EMB)


# speedup vs baseline: 1.2811x; 1.0620x over previous
import functools
import math

import jax
import jax.numpy as jnp
from jax import lax
from jax.experimental import pallas as pl
from jax.experimental.pallas import tpu as pltpu
from jax.experimental.pallas import tpu_sc as plsc

_VOCAB = 100000
_EMB = 128
_SCALE = math.sqrt(float(_EMB))

_B = 4096 * 200
_NW = 32
_BPW = _B // _NW
_C = 128
_NCHUNK = _BPW // _C
_NBF = 4
_NRW = 2
_W = _EMB // 2

_mesh = plsc.VectorSubcoreMesh(core_axis_name="c", subcore_axis_name="s")


@functools.partial(
    pl.kernel,
    mesh=_mesh,
    compiler_params=pltpu.CompilerParams(
        needs_layout_passes=False, use_tc_tiling_on_sc=False
    ),
    out_type=jax.ShapeDtypeStruct((_B, _EMB), jnp.float32),
    scratch_types=[
        pltpu.VMEM((_NCHUNK, _C), jnp.int32),
        pltpu.VMEM((_NBF, _C, _W), jnp.int32),
        pltpu.VMEM((_NRW, _C, _EMB), jnp.float32),
        pltpu.SemaphoreType.DMA,
        pltpu.SemaphoreType.DMA,
    ],
)
def _gather(tokens_hbm, table_hbm, out_hbm, idx_v, pk_v, rows_v, gsem, ssem):
    cid = lax.axis_index("c")
    sid = lax.axis_index("s")
    wid = sid * 2 + cid
    base = wid * _BPW

    pltpu.sync_copy(tokens_hbm.at[wid], idx_v)

    def g_copy(g, b):
        return pltpu.make_async_copy(
            table_hbm.at[idx_v.at[g]], pk_v.at[b], gsem
        )

    def s_copy(g, rs):
        return pltpu.make_async_copy(
            rows_v.at[rs], out_hbm.at[pl.ds(base + g * _C, _C)], ssem
        )

    himask = jnp.full((16,), -65536, jnp.int32)
    lanes = lax.iota(jnp.int32, 16)
    ev_cols = [lanes * 2 + 32 * j for j in range(_EMB // 32)]
    od_cols = [c + 1 for c in ev_cols]

    def expand(bs, rs):
        dst = rows_v.at[rs]

        @plsc.parallel_loop(0, _C, unroll=4)
        def erow(r):
            rr = jnp.full((16,), r, jnp.int32)
            for j in range(_W // 16):
                w = pk_v[bs, r, pl.ds(16 * j, 16)]
                lo = plsc.bitcast(w << 16, jnp.float32) * _SCALE
                hi = plsc.bitcast(w & himask, jnp.float32) * _SCALE
                plsc.store_scatter(dst, [rr, ev_cols[j]], lo)
                plsc.store_scatter(dst, [rr, od_cols[j]], hi)

    def chunk(g, b, wait_s, issue_g):
        rs = b % _NRW
        g_copy(g, b).wait()
        if wait_s:
            s_copy(g - _NRW, rs).wait()
        expand(b, rs)
        s_copy(g, rs).start()
        if issue_g:
            g_copy(g + _NBF, b).start()

    for b in range(_NBF):
        g_copy(b, b).start()

    for b in range(_NBF):
        chunk(b, b, wait_s=(b >= _NRW), issue_g=True)

    def body(i, carry):
        g0 = i * _NBF
        for b in range(_NBF):
            chunk(g0 + b, b, wait_s=True, issue_g=True)
        return carry

    lax.fori_loop(1, _NCHUNK // _NBF - 1, body, 0)

    g0 = _NCHUNK - _NBF
    for b in range(_NBF):
        chunk(g0 + b, b, wait_s=True, issue_g=False)

    for g in range(_NCHUNK - _NRW, _NCHUNK):
        s_copy(g, g % _NRW).wait()


def kernel(tokens, table):
    tbl = lax.bitcast_convert_type(
        table.astype(jnp.bfloat16).reshape(_VOCAB, _W, 2), jnp.int32
    )
    tok = tokens.reshape(_NW, _NCHUNK, _C).astype(jnp.int32)
    out = _gather(tok, tbl)
    return out.reshape(tokens.shape[0], tokens.shape[1], _EMB)


# speedup vs baseline: 2.4528x; 1.9146x over previous
import functools
import math

import jax
import jax.numpy as jnp
from jax import lax
from jax.experimental import pallas as pl
from jax.experimental.pallas import tpu as pltpu
from jax.experimental.pallas import tpu_sc as plsc

_VOCAB = 100000
_EMB = 128
_SCALE = math.sqrt(float(_EMB))

_B = 4096 * 200
_NW = 32
_BPW = _B // _NW
_C = 128
_NCHUNK = _BPW // _C
_NBF = 4
_NRW = 2
_W = _EMB // 2

_mesh = plsc.VectorSubcoreMesh(core_axis_name="c", subcore_axis_name="s")


@functools.partial(
    pl.kernel,
    mesh=_mesh,
    compiler_params=pltpu.CompilerParams(
        needs_layout_passes=False, use_tc_tiling_on_sc=False
    ),
    out_type=jax.ShapeDtypeStruct((_B, _EMB), jnp.float32),
    scratch_types=[
        pltpu.VMEM((_NCHUNK, _C), jnp.int32),
        pltpu.VMEM((_NBF, _C, _W), jnp.int32),
        pltpu.VMEM((_NRW, _C, _EMB), jnp.float32),
        pltpu.SemaphoreType.DMA,
        pltpu.SemaphoreType.DMA,
    ],
)
def _gather(tokens_hbm, table_hbm, out_hbm, idx_v, pk_v, rows_v, gsem, ssem):
    cid = lax.axis_index("c")
    sid = lax.axis_index("s")
    wid = sid * 2 + cid
    base = wid * _BPW

    pltpu.sync_copy(tokens_hbm.at[wid], idx_v)

    def g_copy(g, b):
        return pltpu.make_async_copy(
            table_hbm.at[idx_v.at[g]], pk_v.at[b], gsem
        )

    def s_copy(g, rs):
        return pltpu.make_async_copy(
            rows_v.at[rs], out_hbm.at[pl.ds(base + g * _C, _C)], ssem
        )

    himask = jnp.full((16,), -65536, jnp.int32)
    def expand(bs, rs):
        @plsc.parallel_loop(0, _C, unroll=4)
        def erow(r):
            for j in range(_W // 16):
                w = pk_v[bs, r, pl.ds(16 * j, 16)]
                lo = plsc.bitcast(w << 16, jnp.float32) * _SCALE
                hi = plsc.bitcast(w & himask, jnp.float32) * _SCALE
                rows_v[rs, r, pl.ds(16 * j, 16)] = lo
                rows_v[rs, r, pl.ds(_W + 16 * j, 16)] = hi

    def chunk(g, b, wait_s, issue_g):
        rs = b % _NRW
        g_copy(g, b).wait()
        if wait_s:
            s_copy(g - _NRW, rs).wait()
        expand(b, rs)
        s_copy(g, rs).start()
        if issue_g:
            g_copy(g + _NBF, b).start()

    for b in range(_NBF):
        g_copy(b, b).start()

    for b in range(_NBF):
        chunk(b, b, wait_s=(b >= _NRW), issue_g=True)

    def body(i, carry):
        g0 = i * _NBF
        for b in range(_NBF):
            chunk(g0 + b, b, wait_s=True, issue_g=True)
        return carry

    lax.fori_loop(1, _NCHUNK // _NBF - 1, body, 0)

    g0 = _NCHUNK - _NBF
    for b in range(_NBF):
        chunk(g0 + b, b, wait_s=True, issue_g=False)

    for g in range(_NCHUNK - _NRW, _NCHUNK):
        s_copy(g, g % _NRW).wait()


def _pack_table(table):
    def body(t_ref, o_ref):
        bits = lax.bitcast_convert_type(t_ref[...], jnp.uint32)
        a = bits[:, :_W]
        b = bits[:, _W:]

        def rnd(u):
            return (u + jnp.uint32(0x7FFF) + ((u >> 16) & jnp.uint32(1))) >> 16

        w = (rnd(b) << 16) | rnd(a)
        o_ref[...] = lax.bitcast_convert_type(w, jnp.int32)

    return pl.pallas_call(
        body,
        grid=(100,),
        in_specs=[pl.BlockSpec((_VOCAB // 100, _EMB), lambda i: (i, 0))],
        out_specs=pl.BlockSpec((_VOCAB // 100, _W), lambda i: (i, 0)),
        out_shape=jax.ShapeDtypeStruct((_VOCAB, _W), jnp.int32),
    )(table)


def kernel(tokens, table):
    tbl = _pack_table(table)
    tok = tokens.reshape(_NW, _NCHUNK, _C).astype(jnp.int32)
    out = _gather(tok, tbl)
    return out.reshape(tokens.shape[0], tokens.shape[1], _EMB)


# speedup vs baseline: 2.9487x; 1.2022x over previous
import functools
import math

import jax
import jax.numpy as jnp
from jax import lax
from jax.experimental import pallas as pl
from jax.experimental.pallas import tpu as pltpu
from jax.experimental.pallas import tpu_sc as plsc

_VOCAB = 100000
_EMB = 128
_SCALE = math.sqrt(float(_EMB))

_B = 4096 * 200
_NW = 32
_BPW = _B // _NW
_C = 128
_NCHUNK = _BPW // _C
_NBUF = 5
_GA = 3

_mesh = plsc.VectorSubcoreMesh(core_axis_name="c", subcore_axis_name="s")


@functools.partial(
    pl.kernel,
    mesh=_mesh,
    out_type=jax.ShapeDtypeStruct((_B, _EMB), jnp.float32),
    scratch_types=[
        pltpu.VMEM((_NCHUNK, _C), jnp.int32),
        pltpu.VMEM((_NBUF, _C, _EMB), jnp.float32),
        pltpu.SemaphoreType.DMA,
        pltpu.SemaphoreType.DMA,
    ],
)
def _gather(tokens_hbm, table_hbm, out_hbm, idx_v, rows_v, gsem, ssem):
    cid = lax.axis_index("c")
    sid = lax.axis_index("s")
    wid = sid * 2 + cid
    base = wid * _BPW

    pltpu.sync_copy(tokens_hbm.at[wid], idx_v)

    def g_copy(g, b):
        return pltpu.make_async_copy(
            table_hbm.at[idx_v.at[g]], rows_v.at[b], gsem
        )

    def s_copy(g, b):
        return pltpu.make_async_copy(
            rows_v.at[b], out_hbm.at[pl.ds(base + g * _C, _C)], ssem
        )

    def scale(b):
        def sbody(r, carry):
            for c in range(_EMB // 16):
                sl = pl.ds(c * 16, 16)
                rows_v[b, r, sl] = rows_v[b, r, sl] * _SCALE
            return carry

        lax.fori_loop(0, _C, sbody, 0)

    def chunk(g, b, wait_s, issue_g):
        g_copy(g, b).wait()
        scale(b)
        s_copy(g, b).start()
        if wait_s:
            s_copy(g - (_NBUF - _GA), (b - (_NBUF - _GA)) % _NBUF).wait()
        if issue_g:
            g_copy(g + _GA, (b + _GA) % _NBUF).start()

    for g in range(_GA):
        g_copy(g, g).start()

    for b in range(_NBUF):
        chunk(b, b, wait_s=(b >= _NBUF - _GA), issue_g=True)

    def body(i, carry):
        g0 = i * _NBUF
        for b in range(_NBUF):
            chunk(g0 + b, b, wait_s=True, issue_g=True)
        return carry

    lax.fori_loop(1, _NCHUNK // _NBUF - 1, body, 0)

    g0 = _NCHUNK - _NBUF
    for b in range(_NBUF):
        chunk(g0 + b, b, wait_s=True, issue_g=(b + _GA < _NBUF))

    for g in range(_NCHUNK - (_NBUF - _GA), _NCHUNK):
        s_copy(g, g % _NBUF).wait()


def kernel(tokens, table):
    tok = tokens.reshape(_NW, _NCHUNK, _C).astype(jnp.int32)
    out = _gather(tok, table)
    return out.reshape(tokens.shape[0], tokens.shape[1], _EMB)
